# baseline (speedup 1.0000x reference)
import functools

import jax
import jax.numpy as jnp
from jax import lax
from jax.experimental import pallas as pl
from jax.experimental.pallas import tpu as pltpu
from jax.experimental.pallas import tpu_sc as plsc

_D = 128
_B = 1024
_H = 200
_TOTAL = _B * _H
_NC = 2
_NS = 16
_NW = _NC * _NS
_PER_W = _TOTAL // _NW
_CHUNK = 128
_SUP = 2
_SCHUNK = _SUP * _CHUNK
_NSUP = _PER_W // _SCHUNK
_NBUF = 3

_mesh = plsc.VectorSubcoreMesh(core_axis_name="c", subcore_axis_name="s")


@functools.partial(
    pl.kernel,
    mesh=_mesh,
    out_type=jax.ShapeDtypeStruct((_TOTAL, _D), jnp.float32),
    scratch_types=(
        [pltpu.VMEM((_NSUP * _SUP, _CHUNK), jnp.int32)]
        + [pltpu.VMEM((_SCHUNK, _D), jnp.float32) for _ in range(_NBUF)]
        + [pltpu.SemaphoreType.DMA for _ in range(2 * _NBUF)]
    ),
)
def _gather_kernel(table_hbm, idx_hbm, out_hbm, idx_v, *scratch):
    bufs = scratch[:_NBUF]
    gsem = scratch[_NBUF:2 * _NBUF]
    wsem = scratch[2 * _NBUF:]

    wid = lax.axis_index("s") * _NC + lax.axis_index("c")
    base = wid * _PER_W

    pltpu.sync_copy(idx_hbm.at[wid], idx_v)

    def gather(J, b):
        for h in range(_SUP):
            pltpu.async_copy(table_hbm.at[idx_v.at[_SUP * J + h]],
                             bufs[b].at[pl.ds(h * _CHUNK, _CHUNK)], gsem[b])

    def gather_wait(b):
        for h in range(_SUP):
            pltpu.make_async_copy(table_hbm.at[idx_v.at[0]],
                                  bufs[b].at[pl.ds(h * _CHUNK, _CHUNK)],
                                  gsem[b]).wait()

    def put(J, b):
        pltpu.async_copy(bufs[b], out_hbm.at[pl.ds(base + J * _SCHUNK, _SCHUNK)],
                         wsem[b])

    def put_wait(b):
        pltpu.make_async_copy(bufs[b], out_hbm.at[pl.ds(base, _SCHUNK)],
                              wsem[b]).wait()


    for b in range(_NBUF):
        gather(b, b)

    gather_wait(0)
    put(0, 0)
    gather_wait(1)
    put(1, 1)
    put_wait(0)
    gather(3, 0)
    gather_wait(2)
    put(2, 2)
    put_wait(1)
    gather(4, 1)

    def body(i, carry):
        j0 = _NBUF * (i + 1)
        for k in range(_NBUF):
            b = k
            bprev = (k + 2) % _NBUF
            gather_wait(b)
            put(j0 + k, b)
            put_wait(bprev)
            gather(j0 + k + 2, bprev)
        return carry

    lax.fori_loop(0, 6, body, 0)

    gather_wait(0)
    put(21, 0)
    put_wait(2)
    gather(23, 2)
    gather_wait(1)
    put(22, 1)
    put_wait(0)
    gather(24, 0)
    gather_wait(2)
    put(23, 2)
    put_wait(1)
    gather_wait(0)
    put(24, 0)
    put_wait(2)
    put_wait(0)


def kernel(input_, item_table):
    idx = input_.reshape(-1).astype(jnp.int32).reshape(_NW, _NSUP * _SUP, _CHUNK)
    out = _gather_kernel(item_table, idx)
    return out.reshape(_B, _H, _D)


# speedup vs baseline: 1.0150x; 1.0150x over previous
import functools

import jax
import jax.numpy as jnp
from jax import lax
from jax.experimental import pallas as pl
from jax.experimental.pallas import tpu as pltpu
from jax.experimental.pallas import tpu_sc as plsc

_D = 128
_B = 1024
_H = 200
_TOTAL = _B * _H
_NC = 2
_NS = 16
_NW = _NC * _NS
_PER_W = _TOTAL // _NW
_CHUNK = 128
_SUP = 2
_SCHUNK = _SUP * _CHUNK
_NSUP = _PER_W // _SCHUNK

_mesh = plsc.VectorSubcoreMesh(core_axis_name="c", subcore_axis_name="s")


@functools.partial(
    pl.kernel,
    mesh=_mesh,
    out_type=jax.ShapeDtypeStruct((_TOTAL, _D), jnp.float32),
    scratch_types=(
        [pltpu.VMEM((_NSUP * _SUP, _CHUNK), jnp.int32)]
        + [pltpu.VMEM((_SCHUNK, _D), jnp.float32) for _ in range(2)]
        + [pltpu.SemaphoreType.DMA for _ in range(4)]
    ),
)
def _gather_kernel(table_hbm, idx_hbm, out_hbm, idx_v, *scratch):
    bufs = scratch[:2]
    gsem = scratch[2:4]
    wsem = scratch[4:]

    wid = lax.axis_index("s") * _NC + lax.axis_index("c")
    base = wid * _PER_W

    pltpu.sync_copy(idx_hbm.at[wid], idx_v)

    def gather(J, b):
        for h in range(_SUP):
            pltpu.async_copy(table_hbm.at[idx_v.at[_SUP * J + h]],
                             bufs[b].at[pl.ds(h * _CHUNK, _CHUNK)], gsem[b])

    def gather_wait(b):
        for h in range(_SUP):
            pltpu.make_async_copy(table_hbm.at[idx_v.at[0]],
                                  bufs[b].at[pl.ds(h * _CHUNK, _CHUNK)],
                                  gsem[b]).wait()

    def put(J, b):
        pltpu.async_copy(bufs[b], out_hbm.at[pl.ds(base + J * _SCHUNK, _SCHUNK)],
                         wsem[b])

    def put_wait(b):
        pltpu.make_async_copy(bufs[b], out_hbm.at[pl.ds(base, _SCHUNK)],
                              wsem[b]).wait()

    gather(0, 0)
    gather(1, 1)

    def body(i, carry):
        j0 = 2 * i
        gather_wait(0)
        put(j0, 0)

        @pl.when(j0 + 2 < _NSUP)
        def _():
            put_wait(0)
            gather(j0 + 2, 0)

        gather_wait(1)
        put(j0 + 1, 1)

        @pl.when(j0 + 3 < _NSUP)
        def _():
            put_wait(1)
            gather(j0 + 3, 1)

        return carry

    lax.fori_loop(0, _NSUP // 2, body, 0)

    gather_wait(0)
    put(_NSUP - 1, 0)
    put_wait(0)
    put_wait(1)


def kernel(input_, item_table):
    idx = input_.reshape(-1).astype(jnp.int32).reshape(_NW, _NSUP * _SUP, _CHUNK)
    out = _gather_kernel(item_table, idx)
    return out.reshape(_B, _H, _D)
